# baseline (speedup 1.0000x reference)
import functools
import math

import jax
import jax.numpy as jnp
import numpy as np
from jax import lax
from jax.experimental import pallas as pl
from jax.experimental.pallas import tpu as pltpu
from jax.experimental.pallas import tpu_sc as plsc

_PAD_IDX = 0
_BB = 16
_NC = 2
_NS = 16
_LANES = 16
_CHUNK = 128


@functools.lru_cache(maxsize=None)
def _make_pe(seq: int, d_model: int):
    position = np.arange(seq, dtype=np.float64)[:, None]
    div_term = np.exp(
        np.arange(0, d_model, 2, dtype=np.float64) * -(math.log(10000.0) / d_model)
    )
    pe = np.zeros((seq, d_model), dtype=np.float64)
    pe[:, 0::2] = np.sin(position * div_term)
    pe[:, 1::2] = np.cos(position * div_term)
    return jnp.asarray(pe, dtype=jnp.float32)


def _mask_kernel(tok_ref, mask_ref):
    _, s, bb = tok_ref.shape
    tok_t = tok_ref[0]
    for r in range(bb):
        col = tok_t[:, r : r + 1]
        m = (col != _PAD_IDX).astype(jnp.float32)
        mask_ref[r] = jnp.broadcast_to(m, (s, s))


def _tc_mask(batch_token):
    b, s = batch_token.shape
    nb = b // _BB
    tok3 = batch_token.reshape(nb, _BB, s).transpose(0, 2, 1)
    return pl.pallas_call(
        _mask_kernel,
        grid=(nb,),
        in_specs=[pl.BlockSpec((1, s, _BB), lambda i: (i, 0, 0))],
        out_specs=pl.BlockSpec((_BB, s, s), lambda i: (i, 0, 0)),
        out_shape=jax.ShapeDtypeStruct((b, s, s), jnp.float32),
    )(tok3)


def _sc_embed(tok_flat, table, pe):
    n = tok_flat.shape[0]
    vocab, d = table.shape
    seq = pe.shape[0]
    nw = _NC * _NS
    per_w = n // nw
    n_chunks = per_w // _CHUNK
    vregs_per_row = d // _LANES
    mesh = plsc.VectorSubcoreMesh(core_axis_name="c", subcore_axis_name="s")

    @functools.partial(
        pl.kernel,
        out_type=jax.ShapeDtypeStruct((n, d), jnp.float32),
        mesh=mesh,
        scratch_types=[
            pltpu.VMEM((_CHUNK,), jnp.int32),
            pltpu.VMEM((_CHUNK, d), jnp.float32),
            pltpu.VMEM((seq, d), jnp.float32),
            pltpu.SemaphoreType.DMA,
        ],
    )
    def sc_kernel(tok_hbm, table_hbm, pe_hbm, out_hbm, idx_v, rows_v, pe_v, sem):
        wid = lax.axis_index("s") * _NC + lax.axis_index("c")
        base = wid * per_w
        pltpu.sync_copy(pe_hbm, pe_v)

        def add_pe(pe_off):
            def body(r, _):
                for j in range(vregs_per_row):
                    sl = pl.ds(j * _LANES, _LANES)
                    rows_v[r, sl] = rows_v[r, sl] + pe_v[pe_off + r, sl]
                return 0

            lax.fori_loop(0, _CHUNK, body, 0)

        for c in range(n_chunks):
            off = base + c * _CHUNK
            pltpu.sync_copy(tok_hbm.at[pl.ds(off, _CHUNK)], idx_v)
            pltpu.async_copy(table_hbm.at[idx_v], rows_v, sem).wait()
            add_pe((c * _CHUNK) % seq)
            pltpu.sync_copy(rows_v, out_hbm.at[pl.ds(off, _CHUNK)])

    return sc_kernel(tok_flat, table, pe)


def kernel(batch_token, table):
    b, s = batch_token.shape
    vocab, d = table.shape
    pe = _make_pe(s, d)
    x_flat = _sc_embed(batch_token.reshape(b * s), table, pe)
    mask = _tc_mask(batch_token)
    return (x_flat.reshape(b, s, d), mask)


# speedup vs baseline: 2.7591x; 2.7591x over previous
import functools
import math

import jax
import jax.numpy as jnp
import numpy as np
from jax import lax
from jax.experimental import pallas as pl
from jax.experimental.pallas import tpu as pltpu
from jax.experimental.pallas import tpu_sc as plsc

_PAD_IDX = 0
_BB = 16
_NC = 2
_NS = 16
_CHUNK = 128
_NBUF = 4


@functools.lru_cache(maxsize=None)
def _make_pe(seq: int, d_model: int):
    position = np.arange(seq, dtype=np.float64)[:, None]
    div_term = np.exp(
        np.arange(0, d_model, 2, dtype=np.float64) * -(math.log(10000.0) / d_model)
    )
    pe = np.zeros((seq, d_model), dtype=np.float64)
    pe[:, 0::2] = np.sin(position * div_term)
    pe[:, 1::2] = np.cos(position * div_term)
    return jnp.asarray(pe, dtype=jnp.float32)


def _fused_kernel(table_ref, pe_ref, out_ref):
    vocab, d = table_ref.shape
    seq = pe_ref.shape[0]
    pe = pe_ref[...]
    for v in range(vocab):
        out_ref[v] = pe + jnp.broadcast_to(table_ref[v : v + 1, :], (seq, d))


def _tc_fused(table, pe):
    vocab, d = table.shape
    seq = pe.shape[0]
    fused = pl.pallas_call(
        _fused_kernel,
        out_shape=jax.ShapeDtypeStruct((vocab, seq, d), jnp.float32),
    )(table, pe)
    return fused.reshape(vocab * seq, d)


def _mask_kernel(tok_ref, dep_ref, mask_ref):
    del dep_ref
    _, s, bb = tok_ref.shape
    tok_t = tok_ref[0]
    for r in range(bb):
        col = tok_t[:, r : r + 1]
        m = (col != _PAD_IDX).astype(jnp.float32)
        mask_ref[r] = jnp.broadcast_to(m, (s, s))


def _tc_mask(batch_token, dep_row):
    b, s = batch_token.shape
    nb = b // _BB
    tok3 = batch_token.reshape(nb, _BB, s).transpose(0, 2, 1)
    return pl.pallas_call(
        _mask_kernel,
        grid=(nb,),
        in_specs=[
            pl.BlockSpec((1, s, _BB), lambda i: (i, 0, 0)),
            pl.BlockSpec(dep_row.shape, lambda i: (0, 0)),
        ],
        out_specs=pl.BlockSpec((_BB, s, s), lambda i: (i, 0, 0)),
        out_shape=jax.ShapeDtypeStruct((b, s, s), jnp.float32),
    )(tok3, dep_row)


def _sc_gather(idx_chunks, fused):
    nw, n_chunks, _ = idx_chunks.shape
    _, d = fused.shape
    n = nw * n_chunks * _CHUNK
    per_w = n // nw
    mesh = plsc.VectorSubcoreMesh(core_axis_name="c", subcore_axis_name="s")

    @functools.partial(
        pl.kernel,
        out_type=jax.ShapeDtypeStruct((n, d), jnp.float32),
        mesh=mesh,
        scratch_types=[
            pltpu.VMEM((n_chunks, _CHUNK), jnp.int32),
            pltpu.VMEM((_NBUF, _CHUNK, d), jnp.float32),
        ]
        + [pltpu.SemaphoreType.DMA] * _NBUF,
    )
    def sc_kernel(idx_hbm, fused_hbm, out_hbm, idx_v, rows_v, *sems):
        wid = lax.axis_index("s") * _NC + lax.axis_index("c")
        base = wid * per_w
        pltpu.sync_copy(idx_hbm.at[wid], idx_v)

        gath_d = [None] * n_chunks
        writ_d = [None] * n_chunks

        def s_gath(c):
            k = c % _NBUF
            if c - _NBUF >= 0:
                writ_d[c - _NBUF].wait()
            gath_d[c] = pltpu.async_copy(
                fused_hbm.at[idx_v.at[c]], rows_v.at[k], sems[k]
            )

        def s_writ(c):
            k = c % _NBUF
            gath_d[c].wait()
            writ_d[c] = pltpu.async_copy(
                rows_v.at[k], out_hbm.at[pl.ds(base + c * _CHUNK, _CHUNK)], sems[k]
            )

        for step in range(n_chunks + 1):
            if step < n_chunks:
                s_gath(step)
            if 0 <= step - 1 < n_chunks:
                s_writ(step - 1)
        for c in range(max(0, n_chunks - _NBUF), n_chunks):
            writ_d[c].wait()

    return sc_kernel(idx_chunks, fused)


def kernel(batch_token, table):
    b, s = batch_token.shape
    vocab, d = table.shape
    pe = _make_pe(s, d)
    fused = _tc_fused(table, pe)
    nw = _NC * _NS
    idx2 = batch_token * s + jax.lax.broadcasted_iota(jnp.int32, (b, s), 1)
    idx_chunks = idx2.reshape(nw, (b * s) // (nw * _CHUNK), _CHUNK)
    x_flat = _sc_gather(idx_chunks, fused)
    mask = _tc_mask(batch_token, fused[:1])
    return (x_flat.reshape(b, s, d), mask)


# speedup vs baseline: 3.9271x; 1.4233x over previous
import functools
import math

import jax
import jax.numpy as jnp
import numpy as np
from jax.experimental import pallas as pl

_PAD_IDX = 0
_BB = 16


@functools.lru_cache(maxsize=None)
def _make_pe(seq: int, d_model: int):
    position = np.arange(seq, dtype=np.float64)[:, None]
    div_term = np.exp(
        np.arange(0, d_model, 2, dtype=np.float64) * -(math.log(10000.0) / d_model)
    )
    pe = np.zeros((seq, d_model), dtype=np.float64)
    pe[:, 0::2] = np.sin(position * div_term)
    pe[:, 1::2] = np.cos(position * div_term)
    return jnp.asarray(pe, dtype=jnp.float32)


def _embed_kernel(tok_ref, table_ref, pe_ref, x_ref, mask_ref):
    _, s, bb = tok_ref.shape
    vocab, _ = table_ref.shape
    table = table_ref[...]
    pe = pe_ref[...]
    tok_t = tok_ref[0]
    iota_v = jax.lax.broadcasted_iota(jnp.int32, (1, vocab), 1)
    for r in range(bb):
        col = tok_t[:, r : r + 1]
        onehot = (col == iota_v).astype(jnp.float32)
        x_ref[r] = jnp.dot(onehot, table, preferred_element_type=jnp.float32) + pe
        m = (col != _PAD_IDX).astype(jnp.float32)
        mask_ref[r] = jnp.broadcast_to(m, (s, s))


def kernel(batch_token, table):
    b, s = batch_token.shape
    vocab, d = table.shape
    pe = _make_pe(s, d)
    nb = b // _BB
    tok3 = batch_token.reshape(nb, _BB, s).transpose(0, 2, 1)
    x, mask = pl.pallas_call(
        _embed_kernel,
        grid=(nb,),
        in_specs=[
            pl.BlockSpec((1, s, _BB), lambda i: (i, 0, 0)),
            pl.BlockSpec((vocab, d), lambda i: (0, 0)),
            pl.BlockSpec((s, d), lambda i: (0, 0)),
        ],
        out_specs=[
            pl.BlockSpec((_BB, s, d), lambda i: (i, 0, 0)),
            pl.BlockSpec((_BB, s, s), lambda i: (i, 0, 0)),
        ],
        out_shape=[
            jax.ShapeDtypeStruct((b, s, d), jnp.float32),
            jax.ShapeDtypeStruct((b, s, s), jnp.float32),
        ],
    )(tok3, table, pe)
    return (x, mask)
